# Initial kernel scaffold; baseline (speedup 1.0000x reference)
import functools

import jax
import jax.numpy as jnp
from jax import lax
from jax.experimental import pallas as pl
from jax.experimental.pallas import tpu as pltpu
from jax.experimental.pallas import tpu_sc as plsc

N = 10000
E = 320000
D = 128
EA = 16
NC, NS, L = 2, 16, 16
NW = NC * NS
EPW = E // NW
BC = 80
NCH = EPW // BC
RPS = N // NS
ZB = 125

_mesh = plsc.VectorSubcoreMesh(core_axis_name="c", subcore_axis_name="s",
                               num_cores=NC, num_subcores=NS)


def _bcast(v, lane):
    idx = jnp.full((L,), lane, dtype=jnp.int32)
    return v.at[idx].get(mode="promise_in_bounds")


@functools.partial(
    pl.kernel,
    out_type=jax.ShapeDtypeStruct((NC, N, EA), jnp.float32),
    mesh=_mesh,
    scratch_types=[
        pltpu.VMEM((BC, EA), jnp.float32),
        pltpu.VMEM((BC,), jnp.int32),
        pltpu.VMEM((ZB, EA), jnp.float32),
        pltpu.VMEM_SHARED((N, EA), jnp.float32),
    ],
)
def _sc_seg_ea(ea_hbm, dst_hbm, out_hbm, ea_v, dst_v, z_v, acc):
    c = lax.axis_index("c")
    s = lax.axis_index("s")
    wid = c * NS + s
    zero = jnp.zeros((L,), jnp.float32)

    def zrow(i, _):
        z_v[i, :] = zero
        return 0
    lax.fori_loop(0, ZB, zrow, 0)
    for j in range(RPS // ZB):
        pltpu.sync_copy(z_v, acc.at[pl.ds(s * RPS + j * ZB, ZB)])
    plsc.subcore_barrier()

    def chunk(i, _):
        base = wid * EPW + i * BC
        pltpu.sync_copy(ea_hbm.at[pl.ds(base, BC)], ea_v)
        pltpu.sync_copy(dst_hbm.at[pl.ds(base, BC)], dst_v)
        pltpu.sync_copy(ea_v, acc.at[dst_v], add=True)
        return 0
    lax.fori_loop(0, NCH, chunk, 0)
    plsc.subcore_barrier()
    pltpu.sync_copy(acc.at[pl.ds(s * RPS, RPS)],
                    out_hbm.at[c, pl.ds(s * RPS, RPS)])


@functools.partial(
    pl.kernel,
    out_type=(jax.ShapeDtypeStruct((NC, N, D), jnp.float32),
              jax.ShapeDtypeStruct((NC, N, L), jnp.float32)),
    mesh=_mesh,
    scratch_types=[
        pltpu.VMEM((BC,), jnp.int32),
        pltpu.VMEM((BC,), jnp.int32),
        pltpu.VMEM((BC, D), jnp.float32),
        pltpu.VMEM((BC, D), jnp.float32),
        pltpu.VMEM((BC, D), jnp.float32),
        pltpu.VMEM((BC, D), jnp.float32),
        pltpu.VMEM((BC, L), jnp.float32),
        pltpu.VMEM((D,), jnp.float32),
        pltpu.VMEM((ZB, D), jnp.float32),
        pltpu.VMEM_SHARED((N, D), jnp.float32),
        pltpu.VMEM_SHARED((N, L), jnp.float32),
        pltpu.SemaphoreType.DMA,
        pltpu.SemaphoreType.DMA,
    ],
)
def _sc_edge(xl_hbm, xr_hbm, ee_hbm, src_hbm, dst_hbm, att_hbm,
             outv_hbm, outp_hbm,
             src_v, dst_v, xl_v, xr_v, ee_v, ov_v, op_v, att_v, z_v,
             accv, accp, sem1, sem2):
    c = lax.axis_index("c")
    s = lax.axis_index("s")
    wid = c * NS + s
    zero = jnp.zeros((L,), jnp.float32)

    pltpu.sync_copy(att_hbm, att_v)

    def zrow(i, _):
        for r in range(D // L):
            z_v[i, pl.ds(r * L, L)] = zero
        return 0
    lax.fori_loop(0, ZB, zrow, 0)
    for j in range(RPS // ZB):
        pltpu.sync_copy(z_v, accv.at[pl.ds(s * RPS + j * ZB, ZB)])
        pltpu.sync_copy(z_v.at[pl.ds(0, ZB), pl.ds(0, L)],
                        accp.at[pl.ds(s * RPS + j * ZB, ZB)])
    plsc.subcore_barrier()

    def chunk(i, _):
        base = wid * EPW + i * BC
        pltpu.sync_copy(src_hbm.at[pl.ds(base, BC)], src_v)
        pltpu.sync_copy(dst_hbm.at[pl.ds(base, BC)], dst_v)
        cp1 = pltpu.async_copy(xl_hbm.at[src_v], xl_v, sem1)
        cp2 = pltpu.async_copy(xr_hbm.at[dst_v], xr_v, sem2)
        pltpu.sync_copy(ee_hbm.at[pl.ds(base, BC)], ee_v)
        cp1.wait()
        cp2.wait()

        def edge(e, _):
            t = jnp.zeros((L,), jnp.float32)
            for r in range(D // L):
                xlv = xl_v[e, pl.ds(r * L, L)]
                m = xlv + xr_v[e, pl.ds(r * L, L)] + ee_v[e, pl.ds(r * L, L)]
                m = jnp.maximum(m, 0.2 * m)
                t = t + m * att_v[pl.ds(r * L, L)]
            p = jnp.exp(_bcast(jnp.cumsum(t), L - 1))
            op_v[e, :] = p
            for r in range(D // L):
                ov_v[e, pl.ds(r * L, L)] = p * xl_v[e, pl.ds(r * L, L)]
            return 0
        lax.fori_loop(0, BC, edge, 0)

        pltpu.sync_copy(ov_v, accv.at[dst_v], add=True)
        pltpu.sync_copy(op_v, accp.at[dst_v], add=True)
        return 0
    lax.fori_loop(0, NCH, chunk, 0)
    plsc.subcore_barrier()
    pltpu.sync_copy(accv.at[pl.ds(s * RPS, RPS)],
                    outv_hbm.at[c, pl.ds(s * RPS, RPS)])
    pltpu.sync_copy(accp.at[pl.ds(s * RPS, RPS)],
                    outp_hbm.at[c, pl.ds(s * RPS, RPS)])


_BN = 400
_BE = 512


def _tc_lin_body(x_ref, wl_ref, wr_ref, bl_ref, br_ref, xl_ref, xr_ref):
    xb = x_ref[...]
    xl_ref[...] = (jnp.dot(xb, wl_ref[...], preferred_element_type=jnp.float32)
                   + bl_ref[...][None, :])
    xr_ref[...] = (jnp.dot(xb, wr_ref[...], preferred_element_type=jnp.float32)
                   + br_ref[...][None, :])


def _tc_ee_body(ea_ref, we_ref, ee_ref):
    ee_ref[...] = jnp.dot(ea_ref[...], we_ref[...],
                          preferred_element_type=jnp.float32)


def _tc_final_body(x_ref, xl_ref, xr_ref, acca_ref, accv_ref, accp_ref,
                   we_ref, att_ref, bias_ref, out_ref):
    acca = acca_ref[0] + acca_ref[1]
    cnt = jnp.maximum(acca[:, 11:12], 1.0)
    loop_ee = jnp.dot(acca / cnt, we_ref[...],
                      preferred_element_type=jnp.float32)
    xl = xl_ref[...]
    m = xl + xr_ref[...] + loop_ee
    m = jnp.maximum(m, 0.2 * m)
    s = jnp.sum(m * att_ref[...][None, :], axis=-1, keepdims=True)
    p = jnp.exp(s)
    num = accv_ref[0] + accv_ref[1] + p * xl
    den = (accp_ref[0] + accp_ref[1])[:, 0:1] + p
    out_ref[...] = x_ref[...] + bias_ref[...][None, :] + num / den


def kernel(x, edge_index, edge_attr, Wl, bl, Wr, br, We, att, bias):
    src = edge_index[0]
    dst = edge_index[1]
    ea16 = jnp.concatenate(
        [edge_attr,
         jnp.ones((E, 1), jnp.float32),
         jnp.zeros((E, EA - edge_attr.shape[1] - 1), jnp.float32)], axis=1)
    WeT = jnp.zeros((EA, D), jnp.float32).at[:We.shape[1]].set(We.T)

    acca = _sc_seg_ea(ea16, dst)

    full = lambda arr: pl.BlockSpec(arr.shape, lambda i: (0,) * arr.ndim)
    xl, xr = pl.pallas_call(
        _tc_lin_body,
        grid=(N // _BN,),
        in_specs=[pl.BlockSpec((_BN, D), lambda i: (i, 0)),
                  full(Wl), full(Wr), full(bl), full(br)],
        out_specs=[pl.BlockSpec((_BN, D), lambda i: (i, 0))] * 2,
        out_shape=[jax.ShapeDtypeStruct((N, D), jnp.float32)] * 2,
    )(x, Wl.T, Wr.T, bl, br)

    ee = pl.pallas_call(
        _tc_ee_body,
        grid=(E // _BE,),
        in_specs=[pl.BlockSpec((_BE, EA), lambda i: (i, 0)), full(WeT)],
        out_specs=pl.BlockSpec((_BE, D), lambda i: (i, 0)),
        out_shape=jax.ShapeDtypeStruct((E, D), jnp.float32),
    )(ea16, WeT)

    accv, accp = _sc_edge(xl, xr, ee, src, dst, att)

    out = pl.pallas_call(
        _tc_final_body,
        grid=(N // _BN,),
        in_specs=[pl.BlockSpec((_BN, D), lambda i: (i, 0)),
                  pl.BlockSpec((_BN, D), lambda i: (i, 0)),
                  pl.BlockSpec((_BN, D), lambda i: (i, 0)),
                  pl.BlockSpec((NC, _BN, EA), lambda i: (0, i, 0)),
                  pl.BlockSpec((NC, _BN, D), lambda i: (0, i, 0)),
                  pl.BlockSpec((NC, _BN, L), lambda i: (0, i, 0)),
                  full(WeT), full(att), full(bias)],
        out_specs=pl.BlockSpec((_BN, D), lambda i: (i, 0)),
        out_shape=jax.ShapeDtypeStruct((N, D), jnp.float32),
    )(x, xl, xr, acca, accv, accp, WeT, att, bias)
    return out


# speedup vs baseline: 5.0729x; 5.0729x over previous
import functools

import jax
import jax.numpy as jnp
from jax import lax
from jax.experimental import pallas as pl
from jax.experimental.pallas import tpu as pltpu
from jax.experimental.pallas import tpu_sc as plsc

N = 10000
E = 320000
D = 128
EA = 16
NC, NS, L = 2, 16, 16
NW = NC * NS
EPW = E // NW
BC = 80
NCH = EPW // BC
NP = 10240
RPS = NP // NS
NA = NP // 8
RAS = NA // NS

_mesh = plsc.VectorSubcoreMesh(core_axis_name="c", subcore_axis_name="s",
                               num_cores=NC, num_subcores=NS)


def _lanesum(v):
    lanes = lax.iota(jnp.int32, L)
    for k in (1, 2, 4, 8):
        v = v + v.at[lanes ^ k].get(mode="promise_in_bounds")
    return v


@functools.partial(
    pl.kernel,
    out_type=jax.ShapeDtypeStruct((NC, NA, D), jnp.float32),
    mesh=_mesh,
    scratch_types=[
        pltpu.VMEM((BC, EA), jnp.float32),
        pltpu.VMEM((BC, D), jnp.float32),
        pltpu.VMEM((BC,), jnp.int32),
        pltpu.VMEM((BC,), jnp.int32),
        pltpu.VMEM((8, D), jnp.float32),
        pltpu.VMEM_SHARED((NA, D), jnp.float32),
    ],
)
def _sc_seg_ea(ea_hbm, dst_hbm, out_hbm, ea_v, eaw_v, dst_v, idx2_v, z_v, acc):
    c = lax.axis_index("c")
    s = lax.axis_index("s")
    wid = c * NS + s
    zero = jnp.zeros((L,), jnp.float32)

    def zrow(i, _):
        for r in range(D // L):
            z_v[i, pl.ds(r * L, L)] = zero
        return 0
    lax.fori_loop(0, 8, zrow, 0)

    def zcp(i, _):
        pltpu.sync_copy(z_v, acc.at[pl.ds(s * RAS + i * 8, 8)])
        return 0
    lax.fori_loop(0, RAS // 8, zcp, 0)
    plsc.subcore_barrier()

    def chunk(i, _):
        base = wid * EPW + i * BC
        pltpu.sync_copy(ea_hbm.at[pl.ds(base, BC)], ea_v)
        pltpu.sync_copy(dst_hbm.at[pl.ds(base, BC)], dst_v)

        def group(g, _):
            dst16 = dst_v[pl.ds(g * L, L)]
            idx2_v[pl.ds(g * L, L)] = lax.shift_right_logical(dst16, 3)
            cg16 = jnp.bitwise_and(dst16, 7)
            for j in range(L):
                e = g * L + j
                eav = ea_v[e, :]
                cgv = cg16.at[jnp.full((L,), j, jnp.int32)].get(
                    mode="promise_in_bounds")
                for r in range(D // L):
                    eqf = (1 - jnp.minimum(jnp.abs(cgv - r), 1)
                           ).astype(jnp.float32)
                    eaw_v[e, pl.ds(r * L, L)] = eav * eqf
            return 0
        lax.fori_loop(0, BC // L, group, 0)
        pltpu.sync_copy(eaw_v, acc.at[idx2_v], add=True)
        return 0
    lax.fori_loop(0, NCH, chunk, 0)
    plsc.subcore_barrier()
    pltpu.sync_copy(acc.at[pl.ds(s * RAS, RAS)],
                    out_hbm.at[c, pl.ds(s * RAS, RAS)])


@functools.partial(
    pl.kernel,
    out_type=(jax.ShapeDtypeStruct((NC, NP, D), jnp.float32),
              jax.ShapeDtypeStruct((NC, 128, D), jnp.float32)),
    mesh=_mesh,
    scratch_types=[
        pltpu.VMEM((BC,), jnp.int32),
        pltpu.VMEM((BC,), jnp.int32),
        pltpu.VMEM((BC,), jnp.int32),
        pltpu.VMEM((BC, D), jnp.float32),
        pltpu.VMEM((BC, D), jnp.float32),
        pltpu.VMEM((BC, D), jnp.float32),
        pltpu.VMEM((D,), jnp.float32),
        pltpu.VMEM((8, D), jnp.float32),
        pltpu.VMEM_SHARED((NP, D), jnp.float32),
        pltpu.VMEM_SHARED((128, D), jnp.float32),
        pltpu.SemaphoreType.DMA,
    ],
)
def _sc_edge(xl_hbm, xr_hbm, ee_hbm, src_hbm, dst_hbm, att_hbm,
             outv_hbm, outd_hbm,
             src_v, dst_v, idx2_v, xl_v, xr_v, ee_v, att_v, z_v,
             accv, dacc, sem):
    c = lax.axis_index("c")
    s = lax.axis_index("s")
    wid = c * NS + s
    zero = jnp.zeros((L,), jnp.float32)
    lanes = lax.iota(jnp.int32, L)

    pltpu.sync_copy(att_hbm, att_v)

    def zrow(i, _):
        for r in range(D // L):
            z_v[i, pl.ds(r * L, L)] = zero
        return 0
    lax.fori_loop(0, 8, zrow, 0)

    def zcp(i, _):
        pltpu.sync_copy(z_v, accv.at[pl.ds(s * RPS + i * 8, 8)])
        return 0
    lax.fori_loop(0, RPS // 8, zcp, 0)
    pltpu.sync_copy(z_v, dacc.at[pl.ds(8 * s, 8)])
    plsc.subcore_barrier()

    def chunk(i, _):
        base = wid * EPW + i * BC
        pltpu.sync_copy(src_hbm.at[pl.ds(base, BC)], src_v)
        pltpu.sync_copy(dst_hbm.at[pl.ds(base, BC)], dst_v)
        cp1 = pltpu.async_copy(xl_hbm.at[src_v], xl_v, sem)
        cp2 = pltpu.async_copy(xr_hbm.at[dst_v], xr_v, sem)
        pltpu.sync_copy(ee_hbm.at[pl.ds(base, BC)], ee_v)
        cp1.wait()
        cp2.wait()

        def group(g, _):
            dst16 = dst_v[pl.ds(g * L, L)]
            idx2_v[pl.ds(g * L, L)] = lax.shift_right_logical(dst16, 7)
            col16 = jnp.bitwise_and(dst16, 127)
            for j in range(L):
                e = g * L + j
                t = zero
                for r in range(D // L):
                    m = (xl_v[e, pl.ds(r * L, L)] + xr_v[e, pl.ds(r * L, L)]
                         + ee_v[e, pl.ds(r * L, L)])
                    m = jnp.maximum(m, 0.2 * m)
                    t = t + m * att_v[pl.ds(r * L, L)]
                p = jnp.exp(_lanesum(t))
                colv = col16.at[jnp.full((L,), j, jnp.int32)].get(
                    mode="promise_in_bounds")
                for r in range(D // L):
                    xl_v[e, pl.ds(r * L, L)] = p * xl_v[e, pl.ds(r * L, L)]
                    eqf = (1 - jnp.minimum(jnp.abs(lanes + (r * L) - colv), 1)
                           ).astype(jnp.float32)
                    ee_v[e, pl.ds(r * L, L)] = p * eqf
            return 0
        lax.fori_loop(0, BC // L, group, 0)

        pltpu.sync_copy(xl_v, accv.at[dst_v], add=True)
        pltpu.sync_copy(ee_v, dacc.at[idx2_v], add=True)
        return 0
    lax.fori_loop(0, NCH, chunk, 0)
    plsc.subcore_barrier()
    pltpu.sync_copy(accv.at[pl.ds(s * RPS, RPS)],
                    outv_hbm.at[c, pl.ds(s * RPS, RPS)])
    pltpu.sync_copy(dacc.at[pl.ds(8 * s, 8)], outd_hbm.at[c, pl.ds(8 * s, 8)])


_BN = 400
_BF = 512
_BE = 512


def _tc_lin_body(x_ref, wl_ref, wr_ref, bl_ref, br_ref, xl_ref, xr_ref):
    xb = x_ref[...]
    xl_ref[...] = (jnp.dot(xb, wl_ref[...], preferred_element_type=jnp.float32)
                   + bl_ref[...][None, :])
    xr_ref[...] = (jnp.dot(xb, wr_ref[...], preferred_element_type=jnp.float32)
                   + br_ref[...][None, :])


def _tc_ee_body(ea_ref, we_ref, ee_ref):
    ee_ref[...] = jnp.dot(ea_ref[...], we_ref[...],
                          preferred_element_type=jnp.float32)


def _tc_final_body(x_ref, xl_ref, xr_ref, acca_ref, accv_ref,
                   den0_ref, den1_ref, we_ref, att_ref, bias_ref, out_ref):
    acca = acca_ref[0] + acca_ref[1]
    cnt = jnp.maximum(acca[:, 11:12], 1.0)
    loop_ee = jnp.dot(acca / cnt, we_ref[...],
                      preferred_element_type=jnp.float32)
    xl = xl_ref[...]
    m = xl + xr_ref[...] + loop_ee
    m = jnp.maximum(m, 0.2 * m)
    s = jnp.sum(m * att_ref[...][None, :], axis=-1, keepdims=True)
    p = jnp.exp(s)
    num = accv_ref[0] + accv_ref[1] + p * xl
    den = den0_ref[...] + den1_ref[...] + p
    out_ref[...] = x_ref[...] + bias_ref[...][None, :] + num / den


def kernel(x, edge_index, edge_attr, Wl, bl, Wr, br, We, att, bias):
    src = edge_index[0]
    dst = edge_index[1]
    ed = edge_attr.shape[1]
    ea16 = jnp.concatenate(
        [edge_attr,
         jnp.ones((E, 1), jnp.float32),
         jnp.zeros((E, EA - ed - 1), jnp.float32)], axis=1)
    WeT16 = jnp.zeros((EA, D), jnp.float32).at[:ed].set(We.T)

    acce = _sc_seg_ea(ea16, dst)
    acca = acce.reshape(NC, NP, EA)

    full = lambda arr: pl.BlockSpec(arr.shape, lambda i: (0,) * arr.ndim)
    xl, xr = pl.pallas_call(
        _tc_lin_body,
        grid=(N // _BN,),
        in_specs=[pl.BlockSpec((_BN, D), lambda i: (i, 0)),
                  full(Wl), full(Wr), full(bl), full(br)],
        out_specs=[pl.BlockSpec((_BN, D), lambda i: (i, 0))] * 2,
        out_shape=[jax.ShapeDtypeStruct((N, D), jnp.float32)] * 2,
    )(x, Wl.T, Wr.T, bl, br)

    ee = pl.pallas_call(
        _tc_ee_body,
        grid=(E // _BE,),
        in_specs=[pl.BlockSpec((_BE, EA), lambda i: (i, 0)), full(WeT16)],
        out_specs=pl.BlockSpec((_BE, D), lambda i: (i, 0)),
        out_shape=jax.ShapeDtypeStruct((E, D), jnp.float32),
    )(ea16, WeT16)

    att_dep = att + 0.0 * acce[0, 0]

    accv, denp = _sc_edge(xl, xr, ee, src, dst, att_dep)
    den0 = denp[0].reshape(-1, 1)[:NP]
    den1 = denp[1].reshape(-1, 1)[:NP]

    out = pl.pallas_call(
        _tc_final_body,
        grid=(NP // _BF,),
        in_specs=[pl.BlockSpec((_BF, D), lambda i: (i, 0)),
                  pl.BlockSpec((_BF, D), lambda i: (i, 0)),
                  pl.BlockSpec((_BF, D), lambda i: (i, 0)),
                  pl.BlockSpec((NC, _BF, EA), lambda i: (0, i, 0)),
                  pl.BlockSpec((NC, _BF, D), lambda i: (0, i, 0)),
                  pl.BlockSpec((_BF, 1), lambda i: (i, 0)),
                  pl.BlockSpec((_BF, 1), lambda i: (i, 0)),
                  full(WeT16), full(att), full(bias)],
        out_specs=pl.BlockSpec((_BF, D), lambda i: (i, 0)),
        out_shape=jax.ShapeDtypeStruct((N, D), jnp.float32),
    )(x, xl, xr, acca, accv, den0, den1, WeT16, att, bias)
    return out
